# baseline (speedup 1.0000x reference)
import jax
import jax.numpy as jnp
from jax.experimental import pallas as pl
from jax.experimental.pallas import tpu as pltpu

C = 384
NE = 20
B = 8
H = 24
W = 24
ROWS = H * W
S = 640
G = 32
PBUF = B * S
ADT = jnp.bfloat16

O0 = 24
OM = PBUF - 2 * O0 - 8


def _masks():
    r = jax.lax.broadcasted_iota(jnp.int32, (PBUF, 1), 0)
    t = (r % S) % W
    mask_m = jnp.where(t == (G % W), 0.0, 1.0).astype(ADT)
    mask_p = jnp.where(t == ((G - 1) % W), 0.0, 1.0).astype(ADT)
    ro = (r[O0:O0 + OM] % S)
    valid = jnp.logical_and(ro >= G, ro < G + ROWS)
    return mask_m, mask_p, valid


def _conv(src, wdy, bias, mask_m, mask_p, valid, resid=None):
    am = jnp.roll(src, 1, axis=0) * mask_m
    ap = jnp.roll(src, -1, axis=0) * mask_p
    p = jnp.concatenate([src, am, ap], axis=1)
    acc = jnp.dot(jax.lax.slice(p, (0, 0), (OM, 3 * C)),
                  wdy[0], preferred_element_type=jnp.float32)
    acc = acc + jnp.dot(jax.lax.slice(p, (W, 0), (W + OM, 3 * C)),
                        wdy[1], preferred_element_type=jnp.float32)
    acc = acc + jnp.dot(jax.lax.slice(p, (2 * W, 0), (2 * W + OM, 3 * C)),
                        wdy[2], preferred_element_type=jnp.float32)
    val = acc + bias
    if resid is not None:
        val = val + resid
    return jnp.where(valid, jnp.maximum(val, 0.0), 0.0)


def _store(dst, val):
    dst[pl.ds(0, O0), :] = jnp.zeros((O0, C), ADT)
    dst[pl.ds(O0, OM), :] = val.astype(ADT)
    dst[pl.ds(O0 + OM, PBUF - O0 - OM), :] = jnp.zeros(
        (PBUF - O0 - OM, C), ADT)


def _resblock(h, wk, rb, mask_m, mask_p, valid):
    r1 = _conv(h, wk[0], rb[0][None, :], mask_m, mask_p, valid)
    hin = jax.lax.slice(h, (O0, 0), (O0 + OM, C)).astype(jnp.float32)
    r1b = jnp.concatenate(
        [jnp.zeros((O0, C), ADT), r1.astype(ADT),
         jnp.zeros((PBUF - O0 - OM, C), ADT)], axis=0)
    r2 = _conv(r1b, wk[1], rb[1][None, :], mask_m, mask_p, valid,
               resid=hin)
    return r2


def _body0(xp, wmap_t, projw_t, pb, gum, emb, wk, rb, h_out):
    mask_m, mask_p, valid = _masks()
    xr = xp[...].reshape(B, S, C)
    pooled = jnp.sum(xr, axis=1) * (1.0 / ROWS)
    m = jax.nn.sigmoid(jnp.dot(pooled, wmap_t[...],
                               preferred_element_type=jnp.float32))
    logits = jnp.dot(m, projw_t[...],
                     preferred_element_type=jnp.float32) + pb[...]
    y = logits + gum[...]
    col = jax.lax.broadcasted_iota(jnp.int32, (B, NE), 1)
    ymax = jnp.max(y, axis=1, keepdims=True)
    amin = jnp.min(jnp.where(y == ymax, col, NE), axis=1, keepdims=True)
    oh = (col == amin).astype(jnp.float32)
    zq = jnp.dot(oh, emb[...], preferred_element_type=jnp.float32)
    s = jax.lax.broadcasted_iota(jnp.int32, (S, 1), 0)
    svalid = jnp.logical_and(s >= G, s < G + ROWS).astype(jnp.float32)
    v = xr + zq[:, None, :] * svalid[None, :, :]
    h = v.reshape(PBUF, C).astype(ADT)
    r2 = _resblock(h, wk, rb, mask_m, mask_p, valid)
    _store(h_out, r2)


def _body1(h_in, wk, rb, h_out):
    mask_m, mask_p, valid = _masks()
    r2 = _resblock(h_in[...], wk, rb, mask_m, mask_p, valid)
    _store(h_out, r2)


def _body2(h_in, wk, rb, out):
    mask_m, mask_p, valid = _masks()
    r2 = _resblock(h_in[...], wk, rb, mask_m, mask_p, valid)
    for b in range(B):
        sl = jax.lax.slice(r2, (b * S + G - O0, 0),
                           (b * S + G - O0 + ROWS, C))
        out[b] = sl.reshape(H, W, C)


def _call0(interpret=False):
    return pl.pallas_call(
        _body0,
        out_shape=jax.ShapeDtypeStruct((PBUF, C), ADT),
        interpret=interpret,
    )


def _call1(interpret=False):
    return pl.pallas_call(
        _body1,
        out_shape=jax.ShapeDtypeStruct((PBUF, C), ADT),
        interpret=interpret,
    )


def _call2(interpret=False):
    return pl.pallas_call(
        _body2,
        out_shape=jax.ShapeDtypeStruct((B, H, W, C), jnp.float32),
        interpret=interpret,
    )


def _relayout(w):
    t = jnp.transpose(w.astype(ADT), (0, 3, 4, 2, 1))
    t = jnp.concatenate([t[:, :, 1:2], t[:, :, 0:1], t[:, :, 2:3]], axis=2)
    return t.reshape(2, 3, 3 * C, C)


def _run(x, W_map, proj_W, proj_b, embed, res_w, res_b, gumbel,
         interpret=False):
    xt = jnp.transpose(x, (0, 2, 3, 1)).reshape(B, ROWS, C)
    xp = jnp.pad(xt, ((0, 0), (G, S - G - ROWS), (0, 0))).reshape(PBUF, C)
    wmap_t = W_map[:, :, 0, 0].T
    projw_t = proj_W[:, :, 0, 0].T
    pb = proj_b.reshape(1, NE)
    gum = gumbel[:, :, 0, 0]
    wks = [_relayout(res_w[i]) for i in range(3)]
    h = _call0(interpret)(xp, wmap_t, projw_t, pb, gum, embed,
                          wks[0], res_b[0])
    h = _call1(interpret)(h, wks[1], res_b[1])
    out = _call2(interpret)(h, wks[2], res_b[2])
    return jnp.transpose(out, (0, 3, 1, 2))


def kernel(x, W_map, proj_W, proj_b, embed, res_w, res_b, gumbel):
    return _run(x, W_map, proj_W, proj_b, embed, res_w, res_b, gumbel)


# speedup vs baseline: 1.1368x; 1.1368x over previous
import jax
import jax.numpy as jnp
from jax.experimental import pallas as pl
from jax.experimental.pallas import tpu as pltpu

C = 384
NE = 20
B = 8
H = 24
W = 24
ROWS = H * W
S = 640
G = 32
PBUF = B * S
ADT = jnp.bfloat16

def _edge_masks():
    sidx = jax.lax.broadcasted_iota(jnp.int32, (S, 1), 0)
    mask_m = jnp.where(sidx % W == (G % W), 0.0, 1.0).astype(ADT)
    mask_p = jnp.where(sidx % W == ((G - 1) % W), 0.0, 1.0).astype(ADT)
    return mask_m, mask_p


def _conv_img(slab, wdy, bias, mask_m, mask_p, resid=None):
    am = jnp.roll(slab, 1, axis=0) * mask_m
    ap = jnp.roll(slab, -1, axis=0) * mask_p
    p = jnp.concatenate([slab, am, ap], axis=1)
    acc = jnp.dot(jax.lax.slice(p, (G - W, 0), (G - W + ROWS, 3 * C)),
                  wdy[0], preferred_element_type=jnp.float32)
    acc = acc + jnp.dot(jax.lax.slice(p, (G, 0), (G + ROWS, 3 * C)),
                        wdy[1], preferred_element_type=jnp.float32)
    acc = acc + jnp.dot(jax.lax.slice(p, (G + W, 0), (G + W + ROWS, 3 * C)),
                        wdy[2], preferred_element_type=jnp.float32)
    val = acc + bias
    if resid is not None:
        val = val + resid
    return jnp.maximum(val, 0.0)


def _conv(src, dst, wk, kj, bias, mask_m, mask_p, resid_src=None,
          out_ref=None):
    for b in range(B):
        base = b * S
        slab = src[pl.ds(base, S), :]
        resid = None
        if resid_src is not None:
            resid = resid_src[pl.ds(base + G, ROWS), :].astype(jnp.float32)
        val = _conv_img(slab, wk[kj], bias, mask_m, mask_p, resid)
        if out_ref is None:
            dst[pl.ds(base + G, ROWS), :] = val.astype(ADT)
        else:
            out_ref[b] = val.reshape(H, W, C)


def _body0(xp, wmap_t, projw_t, pb, gum, emb, wk, rb, h_out, r_scr):
    mask_m, mask_p = _edge_masks()
    xr = xp[...].reshape(B, S, C)
    pooled = jnp.sum(xr, axis=1) * (1.0 / ROWS)
    m = jax.nn.sigmoid(jnp.dot(pooled, wmap_t[...],
                               preferred_element_type=jnp.float32))
    logits = jnp.dot(m, projw_t[...],
                     preferred_element_type=jnp.float32) + pb[...]
    y = logits + gum[...]
    col = jax.lax.broadcasted_iota(jnp.int32, (B, NE), 1)
    ymax = jnp.max(y, axis=1, keepdims=True)
    amin = jnp.min(jnp.where(y == ymax, col, NE), axis=1, keepdims=True)
    oh = (col == amin).astype(jnp.float32)
    zq = jnp.dot(oh, emb[...], preferred_element_type=jnp.float32)
    s = jax.lax.broadcasted_iota(jnp.int32, (S, 1), 0)
    svalid = jnp.logical_and(s >= G, s < G + ROWS).astype(jnp.float32)
    v = xr + zq[:, None, :] * svalid[None, :, :]
    h_out[...] = v.reshape(PBUF, C).astype(ADT)
    r_scr[...] = jnp.zeros((PBUF, C), ADT)
    _conv(h_out, r_scr, wk, 0, rb[0][None, :], mask_m, mask_p)
    _conv(r_scr, h_out, wk, 1, rb[1][None, :], mask_m, mask_p,
          resid_src=h_out)


def _body1(h_in, wk, rb, h_out, r_scr):
    mask_m, mask_p = _edge_masks()
    h_out[...] = jnp.zeros((PBUF, C), ADT)
    r_scr[...] = jnp.zeros((PBUF, C), ADT)
    _conv(h_in, r_scr, wk, 0, rb[0][None, :], mask_m, mask_p)
    _conv(r_scr, h_out, wk, 1, rb[1][None, :], mask_m, mask_p,
          resid_src=h_in)


def _body2(h_in, wk, rb, out, r_scr):
    mask_m, mask_p = _edge_masks()
    r_scr[...] = jnp.zeros((PBUF, C), ADT)
    _conv(h_in, r_scr, wk, 0, rb[0][None, :], mask_m, mask_p)
    _conv(r_scr, None, wk, 1, rb[1][None, :], mask_m, mask_p,
          resid_src=h_in, out_ref=out)


def _call0(interpret=False):
    return pl.pallas_call(
        _body0,
        out_shape=jax.ShapeDtypeStruct((PBUF, C), ADT),
        scratch_shapes=[pltpu.VMEM((PBUF, C), ADT)],
        interpret=interpret,
    )


def _call1(interpret=False):
    return pl.pallas_call(
        _body1,
        out_shape=jax.ShapeDtypeStruct((PBUF, C), ADT),
        scratch_shapes=[pltpu.VMEM((PBUF, C), ADT)],
        interpret=interpret,
    )


def _call2(interpret=False):
    return pl.pallas_call(
        _body2,
        out_shape=jax.ShapeDtypeStruct((B, H, W, C), jnp.float32),
        scratch_shapes=[pltpu.VMEM((PBUF, C), ADT)],
        interpret=interpret,
    )


def _relayout(w):
    t = jnp.transpose(w.astype(ADT), (0, 3, 4, 2, 1))
    t = jnp.concatenate([t[:, :, 1:2], t[:, :, 0:1], t[:, :, 2:3]], axis=2)
    return t.reshape(2, 3, 3 * C, C)


def _run(x, W_map, proj_W, proj_b, embed, res_w, res_b, gumbel,
         interpret=False):
    xt = jnp.transpose(x, (0, 2, 3, 1)).reshape(B, ROWS, C)
    xp = jnp.pad(xt, ((0, 0), (G, S - G - ROWS), (0, 0))).reshape(PBUF, C)
    wmap_t = W_map[:, :, 0, 0].T
    projw_t = proj_W[:, :, 0, 0].T
    pb = proj_b.reshape(1, NE)
    gum = gumbel[:, :, 0, 0]
    wks = [_relayout(res_w[i]) for i in range(3)]
    h = _call0(interpret)(xp, wmap_t, projw_t, pb, gum, embed,
                          wks[0], res_b[0])
    h = _call1(interpret)(h, wks[1], res_b[1])
    out = _call2(interpret)(h, wks[2], res_b[2])
    return jnp.transpose(out, (0, 3, 1, 2))


def kernel(x, W_map, proj_W, proj_b, embed, res_w, res_b, gumbel):
    return _run(x, W_map, proj_W, proj_b, embed, res_w, res_b, gumbel)
